# Initial kernel scaffold; baseline (speedup 1.0000x reference)
import jax
import jax.numpy as jnp
from jax.experimental import pallas as pl


def kernel(x, edge_index, edge_attr, W_node, W_edge, W1, b1, W2, b2, ln_scale, ln_bias):
    raise NotImplementedError("write your pallas kernel here")


# speedup vs baseline: 2.4907x; 2.4907x over previous
import functools

import jax
import jax.numpy as jnp
from jax import lax
from jax.experimental import pallas as pl
from jax.experimental.pallas import tpu as pltpu
from jax.experimental.pallas import tpu_sc as plsc

N_NODES = 10000
N_EDGES = 320000
D = 128
ED = 16
NUM_LAYERS = 3

NCORES = 2
NSUB = 16
NWORK = NCORES * NSUB
CHUNK = 128
EPAD = ((N_EDGES + NWORK * CHUNK * 8 - 1) // (NWORK * CHUNK * 8)) \
    * (NWORK * CHUNK * 8)
CPW = EPAD // (NWORK * CHUNK)
NROWS = 10240
ROWS_PER_TILE = NROWS // NSUB
BN = 1000
GRID = N_NODES // BN


def _zero_vmem_2d(ref, nrows, ncols):
    zv = jnp.zeros((16,), jnp.float32)

    def body(r, _):
        for j in range(ncols // 16):
            ref[r, pl.ds(j * 16, 16)] = zv
        return 0

    lax.fori_loop(0, nrows, body, 0)


def _sc_spmm(h, src_p, dst_p):
    mesh = plsc.VectorSubcoreMesh(core_axis_name="c", subcore_axis_name="s")

    @functools.partial(
        pl.kernel,
        out_type=jax.ShapeDtypeStruct((NCORES, NROWS, D), jnp.float32),
        mesh=mesh,
        scratch_types=[
            pltpu.VMEM((CPW, CHUNK), jnp.int32),
            pltpu.VMEM((CPW, CHUNK), jnp.int32),
            pltpu.VMEM((CHUNK, D), jnp.float32),
            pltpu.VMEM_SHARED((NROWS, D), jnp.float32),
            pltpu.SemaphoreType.DMA,
        ],
    )
    def k(h_hbm, src_hbm, dst_hbm, out_hbm, src_v, dst_v, rows_v, s_sh, sem):
        c = lax.axis_index("c")
        s = lax.axis_index("s")
        wid = c * NSUB + s

        pltpu.sync_copy(src_hbm.at[pl.ds(wid * CPW, CPW)], src_v)
        pltpu.sync_copy(dst_hbm.at[pl.ds(wid * CPW, CPW)], dst_v)

        _zero_vmem_2d(rows_v, CHUNK, D)
        for q in range(ROWS_PER_TILE // CHUNK):
            pltpu.sync_copy(
                rows_v, s_sh.at[pl.ds(s * ROWS_PER_TILE + q * CHUNK, CHUNK)]
            )
        plsc.subcore_barrier()

        def body(i, _):
            pltpu.async_copy(h_hbm.at[src_v.at[i]], rows_v, sem).wait()
            pltpu.sync_copy(rows_v, s_sh.at[dst_v.at[i]], add=True)
            return 0

        lax.fori_loop(0, CPW, body, 0)
        plsc.subcore_barrier()

        pltpu.sync_copy(
            s_sh.at[pl.ds(s * ROWS_PER_TILE, ROWS_PER_TILE)],
            out_hbm.at[c, pl.ds(s * ROWS_PER_TILE, ROWS_PER_TILE)],
        )

    return k(h, src_p, dst_p)


def _sc_edge_agg(ea_p, dst_p):
    mesh = plsc.VectorSubcoreMesh(core_axis_name="c", subcore_axis_name="s")

    @functools.partial(
        pl.kernel,
        out_type=jax.ShapeDtypeStruct((NCORES, NROWS, D), jnp.float32),
        mesh=mesh,
        scratch_types=[
            pltpu.VMEM((CPW, CHUNK), jnp.int32),
            pltpu.VMEM((CHUNK, ED), jnp.float32),
            pltpu.VMEM((CHUNK, D), jnp.float32),
            pltpu.VMEM_SHARED((NROWS, D), jnp.float32),
        ],
    )
    def k(ea_hbm, dst_hbm, out_hbm, dst_v, ea_v, wide_v, e_sh):
        c = lax.axis_index("c")
        s = lax.axis_index("s")
        wid = c * NSUB + s

        pltpu.sync_copy(dst_hbm.at[pl.ds(wid * CPW, CPW)], dst_v)

        _zero_vmem_2d(wide_v, CHUNK, D)
        for q in range(ROWS_PER_TILE // CHUNK):
            pltpu.sync_copy(
                wide_v, e_sh.at[pl.ds(s * ROWS_PER_TILE + q * CHUNK, CHUNK)]
            )
        plsc.subcore_barrier()

        def body(i, _):
            base = (wid * CPW + i) * CHUNK
            pltpu.sync_copy(ea_hbm.at[pl.ds(base, CHUNK)], ea_v)

            def cp(r, _):
                wide_v[r, pl.ds(0, 16)] = ea_v[r, pl.ds(0, 16)]
                return 0

            lax.fori_loop(0, CHUNK, cp, 0)
            pltpu.sync_copy(wide_v, e_sh.at[dst_v.at[i]], add=True)
            return 0

        lax.fori_loop(0, CPW, body, 0)
        plsc.subcore_barrier()

        pltpu.sync_copy(
            e_sh.at[pl.ds(s * ROWS_PER_TILE, ROWS_PER_TILE)],
            out_hbm.at[c, pl.ds(s * ROWS_PER_TILE, ROWS_PER_TILE)],
        )

    return k(ea_p, dst_p)


def _mm_body(x_ref, w_ref, o_ref):
    o_ref[...] = jnp.dot(x_ref[...], w_ref[...],
                         preferred_element_type=jnp.float32)


def _tc_node_proj(x, w_node):
    return pl.pallas_call(
        _mm_body,
        grid=(GRID,),
        in_specs=[
            pl.BlockSpec((BN, D), lambda i: (i, 0)),
            pl.BlockSpec((D, D), lambda i: (0, 0)),
        ],
        out_specs=pl.BlockSpec((BN, D), lambda i: (i, 0)),
        out_shape=jax.ShapeDtypeStruct((N_NODES, D), jnp.float32),
    )(x, w_node)


def _layer_body(s_ref, e_ref, h_ref, we_ref, w1_ref, b1_ref, w2_ref, b2_ref,
                g_ref, bb_ref, o_ref):
    sagg = s_ref[0] + s_ref[1]
    eagg = (e_ref[0] + e_ref[1])[:, :ED]
    agg = sagg + jnp.dot(eagg, we_ref[...], preferred_element_type=jnp.float32)
    z = jnp.dot(agg, w1_ref[...], preferred_element_type=jnp.float32) \
        + b1_ref[...]
    u = jnp.dot(jnp.maximum(z, 0.0), w2_ref[...],
                preferred_element_type=jnp.float32) + b2_ref[...]
    t = h_ref[...] + u
    mu = jnp.mean(t, axis=-1, keepdims=True)
    var = jnp.mean((t - mu) * (t - mu), axis=-1, keepdims=True)
    o_ref[...] = (t - mu) * lax.rsqrt(var + 1e-5) * g_ref[...] + bb_ref[...]


def _tc_layer(s2, e2, h, w_e, w1, b1, w2, b2, g, b):
    return pl.pallas_call(
        _layer_body,
        grid=(GRID,),
        in_specs=[
            pl.BlockSpec((NCORES, BN, D), lambda i: (0, i, 0)),
            pl.BlockSpec((NCORES, BN, D), lambda i: (0, i, 0)),
            pl.BlockSpec((BN, D), lambda i: (i, 0)),
            pl.BlockSpec((ED, D), lambda i: (0, 0)),
            pl.BlockSpec((D, 2 * D), lambda i: (0, 0)),
            pl.BlockSpec((1, 2 * D), lambda i: (0, 0)),
            pl.BlockSpec((2 * D, D), lambda i: (0, 0)),
            pl.BlockSpec((1, D), lambda i: (0, 0)),
            pl.BlockSpec((1, D), lambda i: (0, 0)),
            pl.BlockSpec((1, D), lambda i: (0, 0)),
        ],
        out_specs=pl.BlockSpec((BN, D), lambda i: (i, 0)),
        out_shape=jax.ShapeDtypeStruct((N_NODES, D), jnp.float32),
    )(s2, e2, h, w_e, w1, b1, w2, b2, g, b)


def kernel(x, edge_index, edge_attr, W_node, W_edge, W1, b1, W2, b2,
           ln_scale, ln_bias):
    src = edge_index[0].astype(jnp.int32)
    dst = edge_index[1].astype(jnp.int32)
    npad = EPAD - N_EDGES
    src_p = jnp.concatenate(
        [src, jnp.zeros((npad,), jnp.int32)]).reshape(EPAD // CHUNK, CHUNK)
    dst_p = jnp.concatenate(
        [dst, jnp.full((npad,), N_NODES, jnp.int32)]).reshape(
            EPAD // CHUNK, CHUNK)
    ea_p = jnp.concatenate(
        [edge_attr, jnp.zeros((npad, ED), jnp.float32)], axis=0)

    e2 = _sc_edge_agg(ea_p, dst_p)
    h = _tc_node_proj(x, W_node)
    for l in range(NUM_LAYERS):
        s2 = _sc_spmm(h, src_p, dst_p)
        h = _tc_layer(
            s2, e2, h, W_edge[l], W1[l], b1[l].reshape(1, -1), W2[l],
            b2[l].reshape(1, -1), ln_scale[l].reshape(1, -1),
            ln_bias[l].reshape(1, -1))
    return h


# speedup vs baseline: 2.7932x; 1.1214x over previous
import functools

import jax
import jax.numpy as jnp
from jax import lax
from jax.experimental import pallas as pl
from jax.experimental.pallas import tpu as pltpu
from jax.experimental.pallas import tpu_sc as plsc

N_NODES = 10000
N_EDGES = 320000
D = 128
ED = 16
NUM_LAYERS = 3

NCORES = 2
NSUB = 16
NWORK = NCORES * NSUB
CHUNK = 128
EPAD = ((N_EDGES + NWORK * CHUNK * 8 - 1) // (NWORK * CHUNK * 8)) \
    * (NWORK * CHUNK * 8)
CPW = EPAD // (NWORK * CHUNK)
NROWS = 10240
ROWS_PER_TILE = NROWS // NSUB
BN = 1000
GRID = N_NODES // BN


def _zero_vmem_2d(ref, nrows, ncols):
    zv = jnp.zeros((16,), jnp.float32)

    def body(r, _):
        for j in range(ncols // 16):
            ref[r, pl.ds(j * 16, 16)] = zv
        return 0

    lax.fori_loop(0, nrows, body, 0)


def _sc_spmm(h, src_p, dst_p):
    mesh = plsc.VectorSubcoreMesh(core_axis_name="c", subcore_axis_name="s")

    GB = 8
    NG = CPW // GB

    @functools.partial(
        pl.kernel,
        out_type=jax.ShapeDtypeStruct((NCORES, NROWS, D), jnp.float32),
        mesh=mesh,
        scratch_types=[
            pltpu.VMEM((2, GB, CHUNK), jnp.int32),
            pltpu.VMEM((2, GB, CHUNK), jnp.int32),
            [pltpu.VMEM((CHUNK, D), jnp.float32)] * 2,
            pltpu.VMEM_SHARED((NROWS, D), jnp.float32),
            [pltpu.SemaphoreType.DMA] * 2,
            [pltpu.SemaphoreType.DMA] * 2,
        ],
    )
    def k(h_hbm, src_hbm, dst_hbm, out_hbm, src_v, dst_v, rows_v, s_sh,
          gsem, ssem):
        c = lax.axis_index("c")
        s = lax.axis_index("s")
        wid = c * NSUB + s
        base = wid * CPW

        _zero_vmem_2d(rows_v[0], CHUNK, D)
        for q in range(ROWS_PER_TILE // CHUNK):
            pltpu.sync_copy(
                rows_v[0],
                s_sh.at[pl.ds(s * ROWS_PER_TILE + q * CHUNK, CHUNK)]
            )
        plsc.subcore_barrier()

        pltpu.sync_copy(src_hbm.at[pl.ds(base, GB)], src_v.at[0])
        pltpu.sync_copy(dst_hbm.at[pl.ds(base, GB)], dst_v.at[0])
        pltpu.async_copy(h_hbm.at[src_v.at[0, 0]], rows_v[0], gsem[0])
        pltpu.async_copy(h_hbm.at[src_v.at[0, 1]], rows_v[1], gsem[1])

        def step(p, pn, j, refill_blk):
            b = j % 2
            pltpu.make_async_copy(
                h_hbm.at[src_v.at[0, 0]], rows_v[b], gsem[b]).wait()
            pltpu.async_copy(
                rows_v[b], s_sh.at[dst_v.at[p, j]], ssem[b], add=True)
            if refill_blk is not None:
                blk, jj = refill_blk
                pltpu.make_async_copy(
                    rows_v[b], s_sh.at[dst_v.at[0, 0]], ssem[b]).wait()
                pltpu.async_copy(
                    h_hbm.at[src_v.at[blk, jj]], rows_v[b], gsem[b])

        def grp(g, _):
            p = g % 2
            pn = (g + 1) % 2
            pltpu.sync_copy(
                src_hbm.at[pl.ds(base + (g + 1) * GB, GB)], src_v.at[pn])
            pltpu.sync_copy(
                dst_hbm.at[pl.ds(base + (g + 1) * GB, GB)], dst_v.at[pn])
            for j in range(GB):
                step(p, pn, j,
                     (p, j + 2) if j < GB - 2 else (pn, j + 2 - GB))
            return 0

        lax.fori_loop(0, NG - 1, grp, 0)
        pl_ = (NG - 1) % 2
        for j in range(GB):
            step(pl_, None, j,
                 (pl_, j + 2) if j < GB - 2 else None)
        for b in range(2):
            pltpu.make_async_copy(
                rows_v[b], s_sh.at[dst_v.at[0, 0]], ssem[b]).wait()
        plsc.subcore_barrier()

        pltpu.sync_copy(
            s_sh.at[pl.ds(s * ROWS_PER_TILE, ROWS_PER_TILE)],
            out_hbm.at[c, pl.ds(s * ROWS_PER_TILE, ROWS_PER_TILE)],
        )

    return k(h, src_p, dst_p)


def _sc_edge_agg(ea_p, dst_p):
    mesh = plsc.VectorSubcoreMesh(core_axis_name="c", subcore_axis_name="s")

    @functools.partial(
        pl.kernel,
        out_type=jax.ShapeDtypeStruct((NCORES, NROWS, D), jnp.float32),
        mesh=mesh,
        scratch_types=[
            pltpu.VMEM((CPW, CHUNK), jnp.int32),
            pltpu.VMEM((CHUNK, ED), jnp.float32),
            pltpu.VMEM((CHUNK, D), jnp.float32),
            pltpu.VMEM_SHARED((NROWS, D), jnp.float32),
        ],
    )
    def k(ea_hbm, dst_hbm, out_hbm, dst_v, ea_v, wide_v, e_sh):
        c = lax.axis_index("c")
        s = lax.axis_index("s")
        wid = c * NSUB + s

        pltpu.sync_copy(dst_hbm.at[pl.ds(wid * CPW, CPW)], dst_v)

        _zero_vmem_2d(wide_v, CHUNK, D)
        for q in range(ROWS_PER_TILE // CHUNK):
            pltpu.sync_copy(
                wide_v, e_sh.at[pl.ds(s * ROWS_PER_TILE + q * CHUNK, CHUNK)]
            )
        plsc.subcore_barrier()

        def body(i, _):
            base = (wid * CPW + i) * CHUNK
            pltpu.sync_copy(ea_hbm.at[pl.ds(base, CHUNK)], ea_v)

            def cp(r, _):
                wide_v[r, pl.ds(0, 16)] = ea_v[r, pl.ds(0, 16)]
                return 0

            lax.fori_loop(0, CHUNK, cp, 0)
            pltpu.sync_copy(wide_v, e_sh.at[dst_v.at[i]], add=True)
            return 0

        lax.fori_loop(0, CPW, body, 0)
        plsc.subcore_barrier()

        pltpu.sync_copy(
            e_sh.at[pl.ds(s * ROWS_PER_TILE, ROWS_PER_TILE)],
            out_hbm.at[c, pl.ds(s * ROWS_PER_TILE, ROWS_PER_TILE)],
        )

    return k(ea_p, dst_p)


def _mm_body(x_ref, w_ref, o_ref):
    o_ref[...] = jnp.dot(x_ref[...], w_ref[...],
                         preferred_element_type=jnp.float32)


def _tc_node_proj(x, w_node):
    return pl.pallas_call(
        _mm_body,
        grid=(GRID,),
        in_specs=[
            pl.BlockSpec((BN, D), lambda i: (i, 0)),
            pl.BlockSpec((D, D), lambda i: (0, 0)),
        ],
        out_specs=pl.BlockSpec((BN, D), lambda i: (i, 0)),
        out_shape=jax.ShapeDtypeStruct((N_NODES, D), jnp.float32),
    )(x, w_node)


def _layer_body(s_ref, e_ref, h_ref, we_ref, w1_ref, b1_ref, w2_ref, b2_ref,
                g_ref, bb_ref, o_ref):
    sagg = s_ref[0] + s_ref[1]
    eagg = (e_ref[0] + e_ref[1])[:, :ED]
    agg = sagg + jnp.dot(eagg, we_ref[...], preferred_element_type=jnp.float32)
    z = jnp.dot(agg, w1_ref[...], preferred_element_type=jnp.float32) \
        + b1_ref[...]
    u = jnp.dot(jnp.maximum(z, 0.0), w2_ref[...],
                preferred_element_type=jnp.float32) + b2_ref[...]
    t = h_ref[...] + u
    mu = jnp.mean(t, axis=-1, keepdims=True)
    var = jnp.mean((t - mu) * (t - mu), axis=-1, keepdims=True)
    o_ref[...] = (t - mu) * lax.rsqrt(var + 1e-5) * g_ref[...] + bb_ref[...]


def _tc_layer(s2, e2, h, w_e, w1, b1, w2, b2, g, b):
    return pl.pallas_call(
        _layer_body,
        grid=(GRID,),
        in_specs=[
            pl.BlockSpec((NCORES, BN, D), lambda i: (0, i, 0)),
            pl.BlockSpec((NCORES, BN, D), lambda i: (0, i, 0)),
            pl.BlockSpec((BN, D), lambda i: (i, 0)),
            pl.BlockSpec((ED, D), lambda i: (0, 0)),
            pl.BlockSpec((D, 2 * D), lambda i: (0, 0)),
            pl.BlockSpec((1, 2 * D), lambda i: (0, 0)),
            pl.BlockSpec((2 * D, D), lambda i: (0, 0)),
            pl.BlockSpec((1, D), lambda i: (0, 0)),
            pl.BlockSpec((1, D), lambda i: (0, 0)),
            pl.BlockSpec((1, D), lambda i: (0, 0)),
        ],
        out_specs=pl.BlockSpec((BN, D), lambda i: (i, 0)),
        out_shape=jax.ShapeDtypeStruct((N_NODES, D), jnp.float32),
    )(s2, e2, h, w_e, w1, b1, w2, b2, g, b)


def kernel(x, edge_index, edge_attr, W_node, W_edge, W1, b1, W2, b2,
           ln_scale, ln_bias):
    src = edge_index[0].astype(jnp.int32)
    dst = edge_index[1].astype(jnp.int32)
    npad = EPAD - N_EDGES
    src_p = jnp.concatenate(
        [src, jnp.zeros((npad,), jnp.int32)]).reshape(EPAD // CHUNK, CHUNK)
    dst_p = jnp.concatenate(
        [dst, jnp.full((npad,), N_NODES, jnp.int32)]).reshape(
            EPAD // CHUNK, CHUNK)
    ea_p = jnp.concatenate(
        [edge_attr, jnp.zeros((npad, ED), jnp.float32)], axis=0)

    e2 = _sc_edge_agg(ea_p, dst_p)
    x = x + e2[0, 0, D - 1] * 0.0
    h = _tc_node_proj(x, W_node)
    for l in range(NUM_LAYERS):
        s2 = _sc_spmm(h, src_p, dst_p)
        h = _tc_layer(
            s2, e2, h, W_edge[l], W1[l], b1[l].reshape(1, -1), W2[l],
            b2[l].reshape(1, -1), ln_scale[l].reshape(1, -1),
            ln_bias[l].reshape(1, -1))
    return h


# speedup vs baseline: 2.8156x; 1.0080x over previous
import functools

import jax
import jax.numpy as jnp
from jax import lax
from jax.experimental import pallas as pl
from jax.experimental.pallas import tpu as pltpu
from jax.experimental.pallas import tpu_sc as plsc

N_NODES = 10000
N_EDGES = 320000
D = 128
ED = 16
NUM_LAYERS = 3

NCORES = 2
NSUB = 16
NWORK = NCORES * NSUB
CHUNK = 128
EPAD = ((N_EDGES + NWORK * CHUNK * 8 - 1) // (NWORK * CHUNK * 8)) \
    * (NWORK * CHUNK * 8)
CPW = EPAD // (NWORK * CHUNK)
NROWS = 10240
ROWS_PER_TILE = NROWS // NSUB
BN = 1000
GRID = N_NODES // BN


def _zero_vmem_2d(ref, nrows, ncols):
    zv = jnp.zeros((16,), jnp.float32)

    def body(r, _):
        for j in range(ncols // 16):
            ref[r, pl.ds(j * 16, 16)] = zv
        return 0

    lax.fori_loop(0, nrows, body, 0)


def _sc_spmm(h, src_p, dst_p):
    mesh = plsc.VectorSubcoreMesh(core_axis_name="c", subcore_axis_name="s")

    GB = 8
    CPW0 = 128
    CPW1 = 2 * CPW - CPW0

    @functools.partial(
        pl.kernel,
        out_type=jax.ShapeDtypeStruct((NCORES, NROWS, D), jnp.float32),
        mesh=mesh,
        scratch_types=[
            pltpu.VMEM((2, GB, CHUNK), jnp.int32),
            pltpu.VMEM((2, GB, CHUNK), jnp.int32),
            [pltpu.VMEM((CHUNK, D), jnp.float32)] * 2,
            pltpu.VMEM_SHARED((NROWS, D), jnp.float32),
            [pltpu.SemaphoreType.DMA] * 2,
            [pltpu.SemaphoreType.DMA] * 2,
        ],
    )
    def k(h_hbm, src_hbm, dst_hbm, out_hbm, src_v, dst_v, rows_v, s_sh,
          gsem, ssem):
        c = lax.axis_index("c")
        s = lax.axis_index("s")
        base = jnp.where(c == 0, s * CPW0, NSUB * CPW0 + s * CPW1)
        ng = jnp.where(c == 0, CPW0 // GB, CPW1 // GB)

        _zero_vmem_2d(rows_v[0], CHUNK, D)
        for q in range(ROWS_PER_TILE // CHUNK):
            pltpu.sync_copy(
                rows_v[0],
                s_sh.at[pl.ds(s * ROWS_PER_TILE + q * CHUNK, CHUNK)]
            )
        plsc.subcore_barrier()

        pltpu.sync_copy(src_hbm.at[pl.ds(base, GB)], src_v.at[0])
        pltpu.sync_copy(dst_hbm.at[pl.ds(base, GB)], dst_v.at[0])
        pltpu.async_copy(h_hbm.at[src_v.at[0, 0]], rows_v[0], gsem[0])
        pltpu.async_copy(h_hbm.at[src_v.at[0, 1]], rows_v[1], gsem[1])

        def grp(g, _):
            p = g % 2
            pn = (g + 1) % 2
            gnext = jnp.minimum(g + 1, ng - 1)
            pltpu.sync_copy(
                src_hbm.at[pl.ds(base + gnext * GB, GB)], src_v.at[pn])
            pltpu.sync_copy(
                dst_hbm.at[pl.ds(base + gnext * GB, GB)], dst_v.at[pn])
            for j in range(GB):
                b = j % 2
                blk, jj = (p, j + 2) if j < GB - 2 else (pn, j + 2 - GB)
                pltpu.make_async_copy(
                    h_hbm.at[src_v.at[0, 0]], rows_v[b], gsem[b]).wait()
                pltpu.async_copy(
                    rows_v[b], s_sh.at[dst_v.at[p, j]], ssem[b], add=True)
                pltpu.make_async_copy(
                    rows_v[b], s_sh.at[dst_v.at[0, 0]], ssem[b]).wait()
                pltpu.async_copy(
                    h_hbm.at[src_v.at[blk, jj]], rows_v[b], gsem[b])
            return 0

        lax.fori_loop(0, ng, grp, 0)
        for b in range(2):
            pltpu.make_async_copy(
                h_hbm.at[src_v.at[0, 0]], rows_v[b], gsem[b]).wait()
        plsc.subcore_barrier()

        pltpu.sync_copy(
            s_sh.at[pl.ds(s * ROWS_PER_TILE, ROWS_PER_TILE)],
            out_hbm.at[c, pl.ds(s * ROWS_PER_TILE, ROWS_PER_TILE)],
        )

    return k(h, src_p, dst_p)


def _sc_edge_agg(ea_p, dst_p):
    mesh = plsc.VectorSubcoreMesh(core_axis_name="c", subcore_axis_name="s")

    @functools.partial(
        pl.kernel,
        out_type=jax.ShapeDtypeStruct((NCORES, NROWS, D), jnp.float32),
        mesh=mesh,
        scratch_types=[
            pltpu.VMEM((CPW, CHUNK), jnp.int32),
            pltpu.VMEM((CHUNK, ED), jnp.float32),
            pltpu.VMEM((CHUNK, D), jnp.float32),
            pltpu.VMEM_SHARED((NROWS, D), jnp.float32),
        ],
    )
    def k(ea_hbm, dst_hbm, out_hbm, dst_v, ea_v, wide_v, e_sh):
        c = lax.axis_index("c")
        s = lax.axis_index("s")
        wid = c * NSUB + s

        pltpu.sync_copy(dst_hbm.at[pl.ds(wid * CPW, CPW)], dst_v)

        _zero_vmem_2d(wide_v, CHUNK, D)
        for q in range(ROWS_PER_TILE // CHUNK):
            pltpu.sync_copy(
                wide_v, e_sh.at[pl.ds(s * ROWS_PER_TILE + q * CHUNK, CHUNK)]
            )
        plsc.subcore_barrier()

        def body(i, _):
            base = (wid * CPW + i) * CHUNK
            pltpu.sync_copy(ea_hbm.at[pl.ds(base, CHUNK)], ea_v)

            def cp(r, _):
                wide_v[r, pl.ds(0, 16)] = ea_v[r, pl.ds(0, 16)]
                return 0

            lax.fori_loop(0, CHUNK, cp, 0)
            pltpu.sync_copy(wide_v, e_sh.at[dst_v.at[i]], add=True)
            return 0

        lax.fori_loop(0, CPW, body, 0)
        plsc.subcore_barrier()

        pltpu.sync_copy(
            e_sh.at[pl.ds(s * ROWS_PER_TILE, ROWS_PER_TILE)],
            out_hbm.at[c, pl.ds(s * ROWS_PER_TILE, ROWS_PER_TILE)],
        )

    return k(ea_p, dst_p)


def _mm_body(x_ref, w_ref, o_ref):
    o_ref[...] = jnp.dot(x_ref[...], w_ref[...],
                         preferred_element_type=jnp.float32)


def _tc_node_proj(x, w_node):
    return pl.pallas_call(
        _mm_body,
        grid=(GRID,),
        in_specs=[
            pl.BlockSpec((BN, D), lambda i: (i, 0)),
            pl.BlockSpec((D, D), lambda i: (0, 0)),
        ],
        out_specs=pl.BlockSpec((BN, D), lambda i: (i, 0)),
        out_shape=jax.ShapeDtypeStruct((N_NODES, D), jnp.float32),
    )(x, w_node)


def _layer_body(s_ref, e_ref, h_ref, we_ref, w1_ref, b1_ref, w2_ref, b2_ref,
                g_ref, bb_ref, o_ref):
    sagg = s_ref[0] + s_ref[1]
    eagg = (e_ref[0] + e_ref[1])[:, :ED]
    agg = sagg + jnp.dot(eagg, we_ref[...], preferred_element_type=jnp.float32)
    z = jnp.dot(agg, w1_ref[...], preferred_element_type=jnp.float32) \
        + b1_ref[...]
    u = jnp.dot(jnp.maximum(z, 0.0), w2_ref[...],
                preferred_element_type=jnp.float32) + b2_ref[...]
    t = h_ref[...] + u
    mu = jnp.mean(t, axis=-1, keepdims=True)
    var = jnp.mean((t - mu) * (t - mu), axis=-1, keepdims=True)
    o_ref[...] = (t - mu) * lax.rsqrt(var + 1e-5) * g_ref[...] + bb_ref[...]


def _tc_layer(s2, e2, h, w_e, w1, b1, w2, b2, g, b):
    return pl.pallas_call(
        _layer_body,
        grid=(GRID,),
        in_specs=[
            pl.BlockSpec((NCORES, BN, D), lambda i: (0, i, 0)),
            pl.BlockSpec((NCORES, BN, D), lambda i: (0, i, 0)),
            pl.BlockSpec((BN, D), lambda i: (i, 0)),
            pl.BlockSpec((ED, D), lambda i: (0, 0)),
            pl.BlockSpec((D, 2 * D), lambda i: (0, 0)),
            pl.BlockSpec((1, 2 * D), lambda i: (0, 0)),
            pl.BlockSpec((2 * D, D), lambda i: (0, 0)),
            pl.BlockSpec((1, D), lambda i: (0, 0)),
            pl.BlockSpec((1, D), lambda i: (0, 0)),
            pl.BlockSpec((1, D), lambda i: (0, 0)),
        ],
        out_specs=pl.BlockSpec((BN, D), lambda i: (i, 0)),
        out_shape=jax.ShapeDtypeStruct((N_NODES, D), jnp.float32),
    )(s2, e2, h, w_e, w1, b1, w2, b2, g, b)


def kernel(x, edge_index, edge_attr, W_node, W_edge, W1, b1, W2, b2,
           ln_scale, ln_bias):
    src = edge_index[0].astype(jnp.int32)
    dst = edge_index[1].astype(jnp.int32)
    npad = EPAD - N_EDGES
    src_p = jnp.concatenate(
        [src, jnp.zeros((npad,), jnp.int32)]).reshape(EPAD // CHUNK, CHUNK)
    dst_p = jnp.concatenate(
        [dst, jnp.full((npad,), N_NODES, jnp.int32)]).reshape(
            EPAD // CHUNK, CHUNK)
    ea_p = jnp.concatenate(
        [edge_attr, jnp.zeros((npad, ED), jnp.float32)], axis=0)

    e2 = _sc_edge_agg(ea_p, dst_p)
    x = x + e2[0, 0, D - 1] * 0.0
    h = _tc_node_proj(x, W_node)
    for l in range(NUM_LAYERS):
        s2 = _sc_spmm(h, src_p, dst_p)
        h = _tc_layer(
            s2, e2, h, W_edge[l], W1[l], b1[l].reshape(1, -1), W2[l],
            b2[l].reshape(1, -1), ln_scale[l].reshape(1, -1),
            ln_bias[l].reshape(1, -1))
    return h
